# baseline (speedup 1.0000x reference)
import jax
import jax.numpy as jnp
from jax.experimental import pallas as pl

_N = 4096


def _body(zut_ref, zit_ref, pt_ref, a_ref, rel_ref, puit_ref, xui_ref):
    zu_b = zut_ref[...]
    zi_b = zit_ref[...]
    pt = pt_ref[...]
    a = a_ref[...]
    rel = rel_ref[...]
    u0 = jnp.dot(pt[:16, :], zu_b, preferred_element_type=jnp.float32)
    u1 = jnp.dot(pt[16:, :], zu_b, preferred_element_type=jnp.float32)
    t0 = jnp.sum(u0 * zi_b, axis=0, keepdims=True)
    t1 = jnp.sum(u1 * zi_b, axis=0, keepdims=True)
    p = a[:, 0:1] * t0 + a[:, 1:2] * t1
    m = jnp.max(p, axis=0, keepdims=True)
    e = jnp.exp(p - m)
    den = jnp.sum(e, axis=0, keepdims=True)
    num = jnp.sum(e * rel, axis=0, keepdims=True)
    puit_ref[...] = p
    xui_ref[...] = (num / den).reshape(-1)


def kernel(zu, zi, P, A, relations):
    b, d = zu.shape
    r = relations.shape[0]
    zut = zu.T
    zit = zi.T
    pt = jnp.swapaxes(P, 1, 2).reshape(2 * d, d)
    a = A[:, :, 0]
    rel = relations.reshape(r, 1)
    grid = b // _N
    puit, xui = pl.pallas_call(
        _body,
        grid=(grid,),
        in_specs=[
            pl.BlockSpec((d, _N), lambda i: (0, i)),
            pl.BlockSpec((d, _N), lambda i: (0, i)),
            pl.BlockSpec((2 * d, d), lambda i: (0, 0)),
            pl.BlockSpec((r, 2), lambda i: (0, 0)),
            pl.BlockSpec((r, 1), lambda i: (0, 0)),
        ],
        out_specs=[
            pl.BlockSpec((r, _N), lambda i: (0, i)),
            pl.BlockSpec((_N,), lambda i: (i,)),
        ],
        out_shape=[
            jax.ShapeDtypeStruct((r, b), jnp.float32),
            jax.ShapeDtypeStruct((b,), jnp.float32),
        ],
    )(zut, zit, pt, a, rel)
    return (xui, puit.T)


# speedup vs baseline: 1.5404x; 1.5404x over previous
import jax
import jax.numpy as jnp
from jax.experimental import pallas as pl

_N = 8192

_CONTRACT0 = (((0,), (0,)), ((), ()))


def _body(zut_ref, zit_ref, pr_ref, prm_ref, puit_ref, xui_ref):
    zu_b = zut_ref[...]
    zi_b = zit_ref[...]
    pr = pr_ref[...]
    prm = prm_ref[...]
    u0 = jax.lax.dot_general(pr[:16, :], zu_b, _CONTRACT0,
                             preferred_element_type=jnp.float32)
    u1 = jax.lax.dot_general(pr[16:, :], zu_b, _CONTRACT0,
                             preferred_element_type=jnp.float32)
    t0 = jnp.sum(u0 * zi_b, axis=0, keepdims=True)
    t1 = jnp.sum(u1 * zi_b, axis=0, keepdims=True)
    t = jnp.concatenate([t0, t1], axis=0)
    p = jax.lax.dot_general(prm[0:2, :], t, _CONTRACT0,
                            preferred_element_type=jnp.float32)
    m = jnp.max(p, axis=0, keepdims=True)
    e = jnp.exp(p - m)
    nd = jnp.dot(prm[2:4, :], e, preferred_element_type=jnp.float32)
    puit_ref[...] = p
    xui_ref[...] = (nd[0:1, :] / nd[1:2, :]).reshape(-1)


def kernel(zu, zi, P, A, relations):
    b, d = zu.shape
    r = relations.shape[0]
    zut = zu.T
    zit = zi.T
    pr = P.reshape(2 * d, d)
    at = jnp.transpose(A, (1, 2, 0)).reshape(2, r)
    prm = jnp.concatenate(
        [at, relations.reshape(1, r), jnp.ones((1, r), jnp.float32)], axis=0)
    grid = b // _N
    puit, xui = pl.pallas_call(
        _body,
        grid=(grid,),
        in_specs=[
            pl.BlockSpec((d, _N), lambda i: (0, i)),
            pl.BlockSpec((d, _N), lambda i: (0, i)),
            pl.BlockSpec((2 * d, d), lambda i: (0, 0)),
            pl.BlockSpec((4, r), lambda i: (0, 0)),
        ],
        out_specs=[
            pl.BlockSpec((r, _N), lambda i: (0, i)),
            pl.BlockSpec((_N,), lambda i: (i,)),
        ],
        out_shape=[
            jax.ShapeDtypeStruct((r, b), jnp.float32),
            jax.ShapeDtypeStruct((b,), jnp.float32),
        ],
    )(zut, zit, pr, prm)
    return (xui, puit.T)
